# Initial kernel scaffold; baseline (speedup 1.0000x reference)
import jax
import jax.numpy as jnp
from jax import lax
from jax.experimental import pallas as pl
from jax.experimental.pallas import tpu as pltpu
from jax.experimental.pallas import tpu_sc as plsc

N_NODES = 10000
D = 128
E = 320000
N_HOPS_K = 3

NC = 2
NS = 16
NW = NC * NS
EPT = E // NW
C = 80
NCHUNK = EPT // C
RPT = N_NODES // NS
ZR = 125
NZ = RPT // ZR

FLAT = N_NODES * D
WPT = FLAT // NW
CW = 4000
NCW = WPT // CW

_MESH = plsc.VectorSubcoreMesh(
    core_axis_name="c", subcore_axis_name="s", num_cores=NC, num_subcores=NS
)


def _hop_body(agg, rowr, colr, trendr, part,
              row_v, col_v, trend_v, buf, zbuf, acc, sem):
    cid = lax.axis_index("c")
    sid = lax.axis_index("s")
    wid = cid * NS + sid

    def zb(j, carry):
        for k in range(D // 16):
            zbuf[j, pl.ds(k * 16, 16)] = jnp.zeros((16,), jnp.float32)
        return carry
    lax.fori_loop(0, ZR, zb, 0)

    pltpu.sync_copy(rowr.at[wid], row_v)
    pltpu.sync_copy(colr.at[wid], col_v)
    pltpu.sync_copy(trendr.at[wid], trend_v)

    def za(k, carry):
        pltpu.sync_copy(zbuf, acc.at[pl.ds(sid * RPT + k * ZR, ZR)])
        return carry
    lax.fori_loop(0, NZ, za, 0)
    plsc.subcore_barrier()

    def chunk(i, carry):
        pltpu.async_copy(agg.at[row_v.at[i]], buf, sem).wait()
        for j in range(C):
            tb = lax.broadcast(trend_v[i, j], (16,))
            for k in range(D // 16):
                buf[j, pl.ds(k * 16, 16)] = buf[j, pl.ds(k * 16, 16)] * tb
        pltpu.sync_copy(buf, acc.at[col_v.at[i]], add=True)
        return carry
    lax.fori_loop(0, NCHUNK, chunk, 0)

    plsc.subcore_barrier()
    pltpu.sync_copy(acc.at[pl.ds(sid * RPT, RPT)],
                    part.at[cid, pl.ds(sid * RPT, RPT)])


def _combine_body(part, out, b0, b1):
    cid = lax.axis_index("c")
    sid = lax.axis_index("s")
    wid = cid * NS + sid
    base = wid * WPT

    def ck(k, carry):
        off = base + k * CW
        pltpu.sync_copy(part.at[0, pl.ds(off, CW)], b0)
        pltpu.sync_copy(part.at[1, pl.ds(off, CW)], b1)

        def add16(j, c2):
            b0[pl.ds(j * 16, 16)] = b0[pl.ds(j * 16, 16)] + b1[pl.ds(j * 16, 16)]
            return c2
        lax.fori_loop(0, CW // 16, add16, 0)
        pltpu.sync_copy(b0, out.at[pl.ds(off, CW)])
        return carry
    lax.fori_loop(0, NCW, ck, 0)


_hop = pl.kernel(
    _hop_body,
    out_type=jax.ShapeDtypeStruct((NC, N_NODES, D), jnp.float32),
    mesh=_MESH,
    scratch_types=[
        pltpu.VMEM((NCHUNK, C), jnp.int32),
        pltpu.VMEM((NCHUNK, C), jnp.int32),
        pltpu.VMEM((NCHUNK, C), jnp.float32),
        pltpu.VMEM((C, D), jnp.float32),
        pltpu.VMEM((ZR, D), jnp.float32),
        pltpu.VMEM_SHARED((N_NODES, D), jnp.float32),
        pltpu.SemaphoreType.DMA,
    ],
)

_combine = pl.kernel(
    _combine_body,
    out_type=jax.ShapeDtypeStruct((FLAT,), jnp.float32),
    mesh=_MESH,
    scratch_types=[
        pltpu.VMEM((CW,), jnp.float32),
        pltpu.VMEM((CW,), jnp.float32),
    ],
)


def kernel(embed, edge_index, trend):
    row = edge_index[0].astype(jnp.int32).reshape(NW, NCHUNK, C)
    col = edge_index[1].astype(jnp.int32).reshape(NW, NCHUNK, C)
    tr = trend.astype(jnp.float32).reshape(NW, NCHUNK, C)

    embs = [embed]
    agg = embed
    for _ in range(N_HOPS_K):
        part = _hop(agg, row, col, tr)
        agg = _combine(part.reshape(NC, FLAT)).reshape(N_NODES, D)
        embs.append(agg)
    return jnp.stack(embs, axis=1)


# speedup vs baseline: 4.0290x; 4.0290x over previous
import jax
import jax.numpy as jnp
from jax import lax
from jax.experimental import pallas as pl
from jax.experimental.pallas import tpu as pltpu
from jax.experimental.pallas import tpu_sc as plsc

N_NODES = 10000
D = 128
E = 320000
N_HOPS_K = 3

NC = 2
NS = 16
NW = NC * NS
EPT = E // NW
C = 128
NCHUNK = 79
EPAD = NCHUNK * C
RPT = 624
ZB = 16
NZ = RPT // ZB
ZR = 104
TAIL = N_NODES - NS * RPT
TAIL_OFF = NS * RPT

RPC = 312
CTAIL_OFF = NW * RPC

_MESH = plsc.VectorSubcoreMesh(
    core_axis_name="c", subcore_axis_name="s", num_cores=NC, num_subcores=NS
)


def _hop_body(agg, rowr, colr, trendr, part,
              row_v, col_v, trend_v, buf, zbuf, acc, sem):
    cid = lax.axis_index("c")
    sid = lax.axis_index("s")
    wid = cid * NS + sid

    def zb(j, carry):
        for k in range(D // 16):
            zbuf[j, pl.ds(k * 16, 16)] = jnp.zeros((16,), jnp.float32)
        return carry
    lax.fori_loop(0, ZB, zb, 0)

    pltpu.sync_copy(rowr.at[wid], row_v)
    pltpu.sync_copy(colr.at[wid], col_v)
    pltpu.sync_copy(trendr.at[wid], trend_v)

    def za(k, carry):
        pltpu.sync_copy(zbuf, acc.at[pl.ds(sid * RPT + k * ZB, ZB)])
        return carry
    lax.fori_loop(0, NZ, za, 0)

    @pl.when(sid == NS - 1)
    def _():
        pltpu.sync_copy(zbuf.at[pl.ds(0, TAIL)], acc.at[pl.ds(TAIL_OFF, TAIL)])
    plsc.subcore_barrier()

    def chunk(i, carry):
        pltpu.async_copy(agg.at[row_v.at[i]], buf, sem).wait()
        for j16 in range(C // 16):
            t16 = trend_v[i, pl.ds(j16 * 16, 16)]
            for jj in range(16):
                j = j16 * 16 + jj
                tb = lax.broadcast(t16[jj], (16,))
                for k in range(D // 16):
                    buf[j, pl.ds(k * 16, 16)] = buf[j, pl.ds(k * 16, 16)] * tb
        pltpu.sync_copy(buf, acc.at[col_v.at[i]], add=True)
        return carry
    lax.fori_loop(0, NCHUNK, chunk, 0)

    plsc.subcore_barrier()
    pltpu.sync_copy(acc.at[pl.ds(sid * RPT, RPT)],
                    part.at[cid, pl.ds(sid * RPT, RPT)])

    @pl.when(sid == NS - 1)
    def _():
        pltpu.sync_copy(acc.at[pl.ds(TAIL_OFF, TAIL)],
                        part.at[cid, pl.ds(TAIL_OFF, TAIL)])


def _combine_body(part, out, b0, b1):
    cid = lax.axis_index("c")
    sid = lax.axis_index("s")
    wid = cid * NS + sid

    def _sum_rows(nrows, off):
        pltpu.sync_copy(part.at[0, pl.ds(off, nrows)], b0.at[pl.ds(0, nrows)])
        pltpu.sync_copy(part.at[1, pl.ds(off, nrows)], b1.at[pl.ds(0, nrows)])

        def addrow(j, c2):
            for kk in range(D // 16):
                b0[j, pl.ds(kk * 16, 16)] = (
                    b0[j, pl.ds(kk * 16, 16)] + b1[j, pl.ds(kk * 16, 16)])
            return c2
        lax.fori_loop(0, nrows, addrow, 0)
        pltpu.sync_copy(b0.at[pl.ds(0, nrows)], out.at[pl.ds(off, nrows)])

    def ck(k, carry):
        _sum_rows(ZR, wid * RPC + k * ZR)
        return carry
    lax.fori_loop(0, RPC // ZR, ck, 0)

    @pl.when(wid == NW - 1)
    def _():
        _sum_rows(TAIL, CTAIL_OFF)


_hop = pl.kernel(
    _hop_body,
    out_type=jax.ShapeDtypeStruct((NC, N_NODES, D), jnp.float32),
    mesh=_MESH,
    scratch_types=[
        pltpu.VMEM((NCHUNK, C), jnp.int32),
        pltpu.VMEM((NCHUNK, C), jnp.int32),
        pltpu.VMEM((NCHUNK, C), jnp.float32),
        pltpu.VMEM((C, D), jnp.float32),
        pltpu.VMEM((ZB, D), jnp.float32),
        pltpu.VMEM_SHARED((N_NODES, D), jnp.float32),
        pltpu.SemaphoreType.DMA,
    ],
)

_combine = pl.kernel(
    _combine_body,
    out_type=jax.ShapeDtypeStruct((N_NODES, D), jnp.float32),
    mesh=_MESH,
    scratch_types=[
        pltpu.VMEM((ZR, D), jnp.float32),
        pltpu.VMEM((ZR, D), jnp.float32),
    ],
)


def _pad_chunks(x):
    x = x.reshape(NW, EPT)
    x = jnp.pad(x, ((0, 0), (0, EPAD - EPT)))
    return x.reshape(NW, NCHUNK, C)


def kernel(embed, edge_index, trend):
    row = _pad_chunks(edge_index[0].astype(jnp.int32))
    col = _pad_chunks(edge_index[1].astype(jnp.int32))
    tr = _pad_chunks(trend.astype(jnp.float32))

    embs = [embed]
    agg = embed
    for _ in range(N_HOPS_K):
        part = _hop(agg, row, col, tr)
        agg = _combine(part)
        embs.append(agg)
    return jnp.stack(embs, axis=1)
